# baseline (speedup 1.0000x reference)
import functools

import jax
import jax.numpy as jnp
from jax import lax
from jax.experimental import pallas as pl
from jax.experimental.pallas import tpu as pltpu
from jax.experimental.pallas import tpu_sc as plsc

MAX_LEN = 8192
DIM = 1024

_info = plsc.get_sparse_core_info()
_NC, _NS, _L = _info.num_cores, _info.num_subcores, _info.num_lanes
_NW = _NC * _NS
_ROWS_PER_W = MAX_LEN // _NW
_CHUNK = 32
_NCHUNK = _ROWS_PER_W // _CHUNK


def _pe_kernel(clamp_hbm, table_hbm, out_hbm, clamp_v, idx_v, rows_v, sem, osem):
    wid = lax.axis_index("s") * _NC + lax.axis_index("c")
    base = wid * _ROWS_PER_W

    pltpu.sync_copy(clamp_hbm, clamp_v)
    clamp_vec = clamp_v[...]
    clamp_s = clamp_vec[0]
    iota = lax.iota(jnp.int32, _L)

    @pl.when(base + _ROWS_PER_W - 1 <= clamp_s)
    def _identity_slab():
        pltpu.async_copy(table_hbm.at[pl.ds(base, _ROWS_PER_W)],
                         out_hbm.at[pl.ds(base, _ROWS_PER_W)], sem).wait()

    @pl.when(base + _ROWS_PER_W - 1 > clamp_s)
    def _clamped_slab():
        def chunk_body(c, _):
            row0 = base + c * _CHUNK
            for j in range(_CHUNK // _L):
                v = jnp.minimum(iota + (row0 + j * _L), clamp_vec)
                idx_v[pl.ds(j * _L, _L)] = jnp.maximum(v, 0)
            pltpu.async_copy(table_hbm.at[idx_v], rows_v, sem).wait()
            pltpu.async_copy(rows_v, out_hbm.at[pl.ds(row0, _CHUNK)],
                             osem).wait()
            return ()

        lax.fori_loop(0, _NCHUNK, chunk_body, ())


@functools.partial(
    pl.kernel,
    out_type=jax.ShapeDtypeStruct((MAX_LEN, DIM), jnp.float32),
    mesh=plsc.VectorSubcoreMesh(core_axis_name="c", subcore_axis_name="s"),
    scratch_types=[
        pltpu.VMEM((_L,), jnp.int32),
        pltpu.VMEM((_CHUNK,), jnp.int32),
        pltpu.VMEM((_CHUNK, DIM), jnp.float32),
        pltpu.SemaphoreType.DMA,
        pltpu.SemaphoreType.DMA,
    ],
)
def _pe_call(clamp_hbm, table_hbm, out_hbm, clamp_v, idx_v, rows_v, sem, osem):
    _pe_kernel(clamp_hbm, table_hbm, out_hbm, clamp_v, idx_v, rows_v, sem, osem)


def kernel(seq_len, table):
    clamp = jnp.full((_L,), jnp.asarray(seq_len, jnp.int32) - 1, jnp.int32)
    return _pe_call(clamp, table)


# speedup vs baseline: 23.2361x; 23.2361x over previous
import functools

import jax
import jax.numpy as jnp
from jax import lax
from jax.experimental import pallas as pl
from jax.experimental.pallas import tpu as pltpu
from jax.experimental.pallas import tpu_sc as plsc

MAX_LEN = 8192
DIM = 1024

_info = plsc.get_sparse_core_info()
_NC, _NS, _L = _info.num_cores, _info.num_subcores, _info.num_lanes
_NW = _NC * _NS
_ROWS_PER_W = MAX_LEN // _NW
_CHUNK = 32
_NCHUNK = _ROWS_PER_W // _CHUNK


def _pe_kernel(clamp_hbm, table_hbm, out_hbm, clamp_v,
               idx0, idx1, buf0, buf1, gsem0, gsem1, wsem0, wsem1):
    wid = lax.axis_index("s") * _NC + lax.axis_index("c")
    base = wid * _ROWS_PER_W

    pltpu.sync_copy(clamp_hbm, clamp_v)
    clamp_vec = clamp_v[...]
    iota = lax.iota(jnp.int32, _L)

    idx = (idx0, idx1)
    buf = (buf0, buf1)
    gsem = (gsem0, gsem1)
    wsem = (wsem0, wsem1)

    def fill_idx(p, c):
        row0 = base + c * _CHUNK
        for j in range(_CHUNK // _L):
            v = jnp.minimum(iota + (row0 + j * _L), clamp_vec)
            idx[p][pl.ds(j * _L, _L)] = jnp.maximum(v, 0)

    def start_gather(p):
        return pltpu.async_copy(table_hbm.at[idx[p]], buf[p], gsem[p])

    def start_write(p, c):
        row0 = base + c * _CHUNK
        return pltpu.async_copy(buf[p], out_hbm.at[pl.ds(row0, _CHUNK)],
                                wsem[p])

    gh = [None, None]
    wh = [None, None]
    fill_idx(0, 0)
    gh[0] = start_gather(0)
    for c in range(_NCHUNK):
        p = c & 1
        if c + 1 < _NCHUNK:
            q = (c + 1) & 1
            fill_idx(q, c + 1)
            if wh[q] is not None:
                wh[q].wait()
            gh[q] = start_gather(q)
        gh[p].wait()
        wh[p] = start_write(p, c)
    wh[(_NCHUNK - 2) & 1].wait()
    wh[(_NCHUNK - 1) & 1].wait()


@functools.partial(
    pl.kernel,
    out_type=jax.ShapeDtypeStruct((MAX_LEN, DIM), jnp.float32),
    mesh=plsc.VectorSubcoreMesh(core_axis_name="c", subcore_axis_name="s"),
    scratch_types=[
        pltpu.VMEM((_L,), jnp.int32),
        pltpu.VMEM((_CHUNK,), jnp.int32),
        pltpu.VMEM((_CHUNK,), jnp.int32),
        pltpu.VMEM((_CHUNK, DIM), jnp.float32),
        pltpu.VMEM((_CHUNK, DIM), jnp.float32),
        pltpu.SemaphoreType.DMA,
        pltpu.SemaphoreType.DMA,
        pltpu.SemaphoreType.DMA,
        pltpu.SemaphoreType.DMA,
    ],
)
def _pe_call(clamp_hbm, table_hbm, out_hbm, clamp_v,
             idx0, idx1, buf0, buf1, gsem0, gsem1, wsem0, wsem1):
    _pe_kernel(clamp_hbm, table_hbm, out_hbm, clamp_v,
               idx0, idx1, buf0, buf1, gsem0, gsem1, wsem0, wsem1)


def kernel(seq_len, table):
    clamp = jnp.full((_L,), jnp.asarray(seq_len, jnp.int32) - 1, jnp.int32)
    return _pe_call(clamp, table)
